# baseline (speedup 1.0000x reference)
import functools
import math

import jax
import jax.numpy as jnp
from jax import lax
from jax.experimental import pallas as pl
from jax.experimental.pallas import tpu as pltpu
from jax.experimental.pallas import tpu_sc as plsc

D_MODEL = 64
SCALE = math.sqrt(D_MODEL)
NC, NS = 2, 16
NW = NC * NS
T_LEN = 200
VEC = 16
STREAM_SPLITS = ((0, 104), (104, 96))


def _emb_body(idx_hbm, lut_hbm, out_hbm, idx_v, jd_v, rows_v, out_v, sem):
    wid = lax.axis_index("s") * NC + lax.axis_index("c")
    n_total = idx_hbm.shape[0]
    rows_per_w = n_total // NW
    blocks_per_w = rows_per_w // T_LEN
    b_base = wid * blocks_per_w

    def block_body(blk, carry):
        b0 = b_base + blk
        row0 = b0 * T_LEN
        pltpu.sync_copy(idx_hbm.at[pl.ds(row0, T_LEN)], idx_v.at[pl.ds(0, T_LEN)])

        def halve(k, c2):
            sl = pl.ds(k * VEC, VEC)
            jd_v[sl] = idx_v[sl] >> 1
            return c2

        lax.fori_loop(0, T_LEN // VEC, halve, 0, unroll=4)
        svl = pl.ds(T_LEN - VEC, VEC)
        jd_v[svl] = idx_v[svl] >> 1

        copies = []
        for off, ln in STREAM_SPLITS:
            copies.append(
                pltpu.async_copy(
                    lut_hbm.at[jd_v.at[pl.ds(off, ln)]],
                    rows_v.at[pl.ds(off, ln)],
                    sem,
                )
            )
        for c in copies:
            c.wait()

        def pack_body(t, c2):
            half = idx_v[pl.ds(t, VEC)][0] & 1
            base = half * D_MODEL
            for v4 in range(D_MODEL // VEC):
                sl = pl.ds(v4 * VEC, VEC)
                out_v[0, t, sl] = rows_v[t, pl.ds(base + v4 * VEC, VEC)] * SCALE
            return c2

        lax.fori_loop(0, T_LEN, pack_body, 0, unroll=2)
        pltpu.sync_copy(out_v, out_hbm.at[pl.ds(b0, 1)])
        return carry

    lax.fori_loop(0, blocks_per_w, block_body, 0)


def kernel(x, lut):
    b, t = x.shape
    n = b * t
    idx = jnp.minimum(x, lut.shape[0] - 1).reshape(n).astype(jnp.int32)
    lutp = lut.reshape(lut.shape[0] // 2, 2 * D_MODEL)
    mesh = plsc.VectorSubcoreMesh(
        core_axis_name="c", subcore_axis_name="s",
        num_cores=NC, num_subcores=NS,
    )
    run = pl.kernel(
        _emb_body,
        out_type=jax.ShapeDtypeStruct((b, t, D_MODEL), jnp.float32),
        mesh=mesh,
        scratch_types=[
            pltpu.VMEM((T_LEN + VEC,), jnp.int32),
            pltpu.VMEM((T_LEN,), jnp.int32),
            pltpu.VMEM((T_LEN, 2 * D_MODEL), jnp.float32),
            pltpu.VMEM((1, T_LEN, D_MODEL), jnp.float32),
            pltpu.SemaphoreType.DMA,
        ],
        compiler_params=pltpu.CompilerParams(use_tc_tiling_on_sc=True),
    )
    return run(idx, lutp)


# speedup vs baseline: 1.6350x; 1.6350x over previous
import functools
import math

import jax
import jax.numpy as jnp
from jax import lax
from jax.experimental import pallas as pl
from jax.experimental.pallas import tpu as pltpu
from jax.experimental.pallas import tpu_sc as plsc

D_MODEL = 64
SCALE = math.sqrt(D_MODEL)
NC, NS = 2, 16
NW = NC * NS
CHUNK = 1024
SUB = 128
VEC = 16


def _emb_body(idx_hbm, lut_hbm, out_hbm, idx_v, rows_v, sem):
    wid = lax.axis_index("s") * NC + lax.axis_index("c")
    n_total = idx_hbm.shape[0]
    per_w = n_total // NW
    n_chunks = per_w // CHUNK
    base = wid * per_w

    def chunk_body(i, carry):
        row0 = base + i * CHUNK
        pltpu.sync_copy(idx_hbm.at[pl.ds(row0, CHUNK)], idx_v)
        copies = []
        for j in range(CHUNK // SUB):
            copies.append(
                pltpu.async_copy(
                    lut_hbm.at[idx_v.at[pl.ds(j * SUB, SUB)]],
                    rows_v.at[pl.ds(j * SUB, SUB)],
                    sem,
                )
            )
        for c in copies:
            c.wait()

        def scale_row(r, c2):
            for v in range(D_MODEL // VEC):
                sl = pl.ds(v * VEC, VEC)
                rows_v[r, sl] = rows_v[r, sl] * SCALE
            return c2

        lax.fori_loop(0, CHUNK, scale_row, 0, unroll=4)
        pltpu.sync_copy(rows_v, out_hbm.at[pl.ds(row0, CHUNK)])
        return carry

    lax.fori_loop(0, n_chunks, chunk_body, 0)


def kernel(x, lut):
    b, t = x.shape
    n = b * t
    idx = jnp.minimum(x, lut.shape[0] - 1).reshape(n).astype(jnp.int32)
    mesh = plsc.VectorSubcoreMesh(
        core_axis_name="c", subcore_axis_name="s",
        num_cores=NC, num_subcores=NS,
    )
    run = pl.kernel(
        _emb_body,
        out_type=jax.ShapeDtypeStruct((n, D_MODEL), jnp.float32),
        mesh=mesh,
        scratch_types=[
            pltpu.VMEM((CHUNK,), jnp.int32),
            pltpu.VMEM((CHUNK, D_MODEL), jnp.float32),
            pltpu.SemaphoreType.DMA,
        ],
        compiler_params=pltpu.CompilerParams(use_tc_tiling_on_sc=False),
    )
    out = run(idx, lut)
    return out.reshape(b, t, D_MODEL)


# speedup vs baseline: 1.7088x; 1.0451x over previous
import functools
import math

import jax
import jax.numpy as jnp
from jax import lax
from jax.experimental import pallas as pl
from jax.experimental.pallas import tpu as pltpu
from jax.experimental.pallas import tpu_sc as plsc

D_MODEL = 64
SCALE = math.sqrt(D_MODEL)
NC, NS = 2, 16
NW = NC * NS
CHUNK = 512
SUB = 128
VEC = 16


def _emb_body(idx_hbm, lut_hbm, out_hbm, i0_v, i1_v, r0_v, r1_v, sem0, sem1):
    idxs = (i0_v, i1_v)
    rows = (r0_v, r1_v)
    sems = (sem0, sem1)
    wid = lax.axis_index("s") * NC + lax.axis_index("c")
    n_total = idx_hbm.shape[0]
    per_w = n_total // NW
    n_pairs = per_w // (2 * CHUNK)
    base = wid * per_w

    def fire(buf, chunk_i):
        row0 = base + chunk_i * CHUNK
        pltpu.sync_copy(idx_hbm.at[pl.ds(row0, CHUNK)], idxs[buf])
        for j in range(CHUNK // SUB):
            pltpu.async_copy(
                lut_hbm.at[idxs[buf].at[pl.ds(j * SUB, SUB)]],
                rows[buf].at[pl.ds(j * SUB, SUB)],
                sems[buf],
            )

    def finish(buf, chunk_i):
        row0 = base + chunk_i * CHUNK
        for j in range(CHUNK // SUB):
            pltpu.make_async_copy(
                lut_hbm.at[idxs[buf].at[pl.ds(j * SUB, SUB)]],
                rows[buf].at[pl.ds(j * SUB, SUB)],
                sems[buf],
            ).wait()

        def scale_row(r, c2):
            for v in range(D_MODEL // VEC):
                sl = pl.ds(v * VEC, VEC)
                rows[buf][r, sl] = rows[buf][r, sl] * SCALE
            return c2

        lax.fori_loop(0, CHUNK, scale_row, 0, unroll=4)
        pltpu.sync_copy(rows[buf], out_hbm.at[pl.ds(row0, CHUNK)])

    fire(0, 0)

    def pair_body(k, carry):
        fire(1, 2 * k + 1)
        finish(0, 2 * k)

        @pl.when(k + 1 < n_pairs)
        def _():
            fire(0, 2 * k + 2)

        finish(1, 2 * k + 1)
        return carry

    lax.fori_loop(0, n_pairs, pair_body, 0)


def kernel(x, lut):
    b, t = x.shape
    n = b * t
    idx = jnp.minimum(x, lut.shape[0] - 1).reshape(n).astype(jnp.int32)
    mesh = plsc.VectorSubcoreMesh(
        core_axis_name="c", subcore_axis_name="s",
        num_cores=NC, num_subcores=NS,
    )
    run = pl.kernel(
        _emb_body,
        out_type=jax.ShapeDtypeStruct((n, D_MODEL), jnp.float32),
        mesh=mesh,
        scratch_types=[
            pltpu.VMEM((CHUNK,), jnp.int32),
            pltpu.VMEM((CHUNK,), jnp.int32),
            pltpu.VMEM((CHUNK, D_MODEL), jnp.float32),
            pltpu.VMEM((CHUNK, D_MODEL), jnp.float32),
            pltpu.SemaphoreType.DMA,
            pltpu.SemaphoreType.DMA,
        ],
        compiler_params=pltpu.CompilerParams(use_tc_tiling_on_sc=False),
    )
    out = run(idx, lut)
    return out.reshape(b, t, D_MODEL)
